# baseline (speedup 1.0000x reference)
import functools

import jax
import jax.numpy as jnp
from jax import lax
from jax.experimental import pallas as pl
from jax.experimental.pallas import tpu as pltpu
from jax.experimental.pallas import tpu_sc as plsc

TM = 256
HK = 1024


def _routing_metadata(rand_maps, T, E, tm):
    perm = jnp.argsort(rand_maps, stable=True).astype(jnp.int32)
    counts = jnp.bincount(rand_maps, length=E)
    ends = jnp.cumsum(counts).astype(jnp.int32)
    starts = (ends - counts).astype(jnp.int32)
    first_tile = (starts // tm).astype(jnp.int32)
    ntiles = jnp.where(counts > 0,
                       (ends + tm - 1) // tm - first_tile, 0).astype(jnp.int32)
    return perm, first_tile, ntiles, starts, ends


def _sc_gather_rows(table, idx):
    Tr, Cc = table.shape
    info = plsc.get_sparse_core_info()
    nw = info.num_cores * info.num_subcores
    bpw = Tr // nw
    mesh = plsc.VectorSubcoreMesh(core_axis_name="c", subcore_axis_name="s")

    @functools.partial(
        pl.kernel,
        mesh=mesh,
        out_type=jax.ShapeDtypeStruct((Tr, Cc), table.dtype),
        scratch_types=[
            pltpu.VMEM((bpw,), jnp.int32),
            pltpu.VMEM((bpw, Cc), table.dtype),
            pltpu.SemaphoreType.DMA,
        ],
    )
    def k(table_hbm, idx_hbm, out_hbm, idx_v, rows_v, sem):
        wid = lax.axis_index("s") * info.num_cores + lax.axis_index("c")
        base = wid * bpw
        pltpu.sync_copy(idx_hbm.at[pl.ds(base, bpw)], idx_v)
        pltpu.async_copy(table_hbm.at[idx_v], rows_v, sem).wait()
        pltpu.sync_copy(rows_v, out_hbm.at[pl.ds(base, bpw)])

    return k(table, idx)


def _sc_scatter_rows(rows, idx):
    Tr, Cc = rows.shape
    info = plsc.get_sparse_core_info()
    nw = info.num_cores * info.num_subcores
    bpw = Tr // nw
    mesh = plsc.VectorSubcoreMesh(core_axis_name="c", subcore_axis_name="s")

    @functools.partial(
        pl.kernel,
        mesh=mesh,
        out_type=jax.ShapeDtypeStruct((Tr, Cc), rows.dtype),
        scratch_types=[
            pltpu.VMEM((bpw,), jnp.int32),
            pltpu.VMEM((bpw, Cc), rows.dtype),
            pltpu.SemaphoreType.DMA,
        ],
    )
    def k(rows_hbm, idx_hbm, out_hbm, idx_v, rows_v, sem):
        wid = lax.axis_index("s") * info.num_cores + lax.axis_index("c")
        base = wid * bpw
        pltpu.sync_copy(idx_hbm.at[pl.ds(base, bpw)], idx_v)
        pltpu.sync_copy(rows_hbm.at[pl.ds(base, bpw)], rows_v)
        pltpu.async_copy(rows_v, out_hbm.at[idx_v], sem).wait()

    return k(rows, idx)


def _ffn_body(ft_ref, nt_ref, ws_ref, we_ref,
              xs_ref, wia_ref, wib_ref, bi_ref, wga_ref, wgb_ref, bg_ref,
              woa_ref, wob_ref, bo_ref, out_ref):
    h = pl.program_id(0)
    g = pl.program_id(1)
    nh = pl.num_programs(0)
    ws = ws_ref[g]
    we = we_ref[g]
    ft = ft_ref[g]
    HH = HK // 2
    halves = ((wia_ref[0], wga_ref[0], woa_ref[0].astype(jnp.bfloat16), 0),
              (wib_ref[0], wgb_ref[0], wob_ref[0].astype(jnp.bfloat16), HH))

    def tile_step(i, carry):
        t = ft + i
        rows = t * TM + lax.broadcasted_iota(jnp.int32, (TM, 1), 0)
        mask = (rows >= ws) & (rows < we)
        x = xs_ref[pl.ds(t * TM, TM), :]
        part = jnp.zeros((TM, x.shape[1]), jnp.float32)
        for wi, wg, wo, off in halves:
            proj = jnp.dot(x, wi, preferred_element_type=jnp.float32,
                           precision=lax.Precision.DEFAULT)
            proj = proj + bi_ref[pl.ds(g, 1), pl.ds(h * HK + off, HH)]
            gate = jnp.dot(x, wg, preferred_element_type=jnp.float32,
                           precision=lax.Precision.DEFAULT)
            gate = gate + bg_ref[pl.ds(g, 1), pl.ds(h * HK + off, HH)]
            u = gate * lax.logistic(gate) * proj
            um = jnp.where(mask, u, 0.0).astype(jnp.bfloat16)
            part = part + jnp.dot(um, wo, preferred_element_type=jnp.float32,
                                  precision=lax.Precision.DEFAULT)
        sl = pl.ds(t * TM, TM)
        prev = out_ref[sl, :]

        @pl.when(h == 0)
        def _():
            out_ref[sl, :] = jnp.where(mask, part, prev)

        @pl.when(jnp.logical_and(h > 0, h < nh - 1))
        def _():
            out_ref[sl, :] = jnp.where(mask, prev + part, prev)

        @pl.when(h == nh - 1)
        def _():
            y = prev + part + bo_ref[pl.ds(g, 1), :]
            out_ref[sl, :] = jnp.where(mask, y, prev)

        return carry

    lax.fori_loop(0, nt_ref[g], tile_step, 0)


def _grouped_ffn(xs, Wi, bi, Wg, bg, Wo, bo, ft, ntl, ws, we):
    T, C = xs.shape
    E, _, H = Wi.shape
    nh = H // HK

    ys = pl.pallas_call(
        _ffn_body,
        grid_spec=pltpu.PrefetchScalarGridSpec(
            num_scalar_prefetch=4,
            grid=(nh, E),
            in_specs=[
                pl.BlockSpec((T, C), lambda h, g, *_: (0, 0)),
                pl.BlockSpec((1, C, HK // 2), lambda h, g, *_: (g, 0, 2 * h)),
                pl.BlockSpec((1, C, HK // 2), lambda h, g, *_: (g, 0, 2 * h + 1)),
                pl.BlockSpec((E, H), lambda h, g, *_: (0, 0)),
                pl.BlockSpec((1, C, HK // 2), lambda h, g, *_: (g, 0, 2 * h)),
                pl.BlockSpec((1, C, HK // 2), lambda h, g, *_: (g, 0, 2 * h + 1)),
                pl.BlockSpec((E, H), lambda h, g, *_: (0, 0)),
                pl.BlockSpec((1, HK // 2, C), lambda h, g, *_: (g, 2 * h, 0)),
                pl.BlockSpec((1, HK // 2, C), lambda h, g, *_: (g, 2 * h + 1, 0)),
                pl.BlockSpec((E, C), lambda h, g, *_: (0, 0)),
            ],
            out_specs=pl.BlockSpec((T, C), lambda h, g, *_: (0, 0)),
        ),
        out_shape=jax.ShapeDtypeStruct((T, C), jnp.float32),
        compiler_params=pltpu.CompilerParams(
            dimension_semantics=("arbitrary", "arbitrary")),
    )(ft, ntl, ws, we, xs, Wi, Wi, bi, Wg, Wg, bg, Wo, Wo, bo)
    return ys


def kernel(x, Wi, bi, Wg, bg, Wo, bo, rand_maps):
    B, T_, C = x.shape
    E = Wi.shape[0]
    T = B * T_
    xf = x.reshape(T, C)

    perm, ft, ntl, ws, we = _routing_metadata(rand_maps, T, E, TM)

    xs = _sc_gather_rows(xf, perm)
    ys = _grouped_ffn(xs, Wi, bi, Wg, bg, Wo, bo, ft, ntl, ws, we)
    outf = _sc_scatter_rows(ys, perm)
    return outf.reshape(B, T_, C)


# speedup vs baseline: 1.0202x; 1.0202x over previous
import functools

import jax
import jax.numpy as jnp
from jax import lax
from jax.experimental import pallas as pl
from jax.experimental.pallas import tpu as pltpu
from jax.experimental.pallas import tpu_sc as plsc

TM = 256
HK = 1024


def _routing_metadata(rand_maps, T, E, tm):
    oh = (rand_maps[:, None] == jnp.arange(E, dtype=rand_maps.dtype)[None, :])
    cnt = jnp.cumsum(oh.astype(jnp.int32), axis=0)
    counts = cnt[-1]
    ends = jnp.cumsum(counts).astype(jnp.int32)
    starts = (ends - counts).astype(jnp.int32)
    rank = (starts[rand_maps]
            + jnp.take_along_axis(cnt, rand_maps[:, None], axis=1)[:, 0] - 1)
    rank = rank.astype(jnp.int32)
    first_tile = (starts // tm).astype(jnp.int32)
    ntiles = jnp.where(counts > 0,
                       (ends + tm - 1) // tm - first_tile, 0).astype(jnp.int32)
    return rank, first_tile, ntiles, starts, ends


def _sc_gather_rows(table, idx):
    Tr, Cc = table.shape
    info = plsc.get_sparse_core_info()
    nw = info.num_cores * info.num_subcores
    bpw = Tr // nw
    mesh = plsc.VectorSubcoreMesh(core_axis_name="c", subcore_axis_name="s")

    @functools.partial(
        pl.kernel,
        mesh=mesh,
        out_type=jax.ShapeDtypeStruct((Tr, Cc), table.dtype),
        scratch_types=[
            pltpu.VMEM((bpw,), jnp.int32),
            pltpu.VMEM((bpw, Cc), table.dtype),
            pltpu.SemaphoreType.DMA,
        ],
    )
    def k(table_hbm, idx_hbm, out_hbm, idx_v, rows_v, sem):
        wid = lax.axis_index("s") * info.num_cores + lax.axis_index("c")
        base = wid * bpw
        pltpu.sync_copy(idx_hbm.at[pl.ds(base, bpw)], idx_v)
        pltpu.async_copy(table_hbm.at[idx_v], rows_v, sem).wait()
        pltpu.sync_copy(rows_v, out_hbm.at[pl.ds(base, bpw)])

    return k(table, idx)


def _sc_scatter_rows(rows, idx):
    Tr, Cc = rows.shape
    info = plsc.get_sparse_core_info()
    nw = info.num_cores * info.num_subcores
    bpw = Tr // nw
    mesh = plsc.VectorSubcoreMesh(core_axis_name="c", subcore_axis_name="s")

    @functools.partial(
        pl.kernel,
        mesh=mesh,
        out_type=jax.ShapeDtypeStruct((Tr, Cc), rows.dtype),
        scratch_types=[
            pltpu.VMEM((bpw,), jnp.int32),
            pltpu.VMEM((bpw, Cc), rows.dtype),
            pltpu.SemaphoreType.DMA,
        ],
    )
    def k(rows_hbm, idx_hbm, out_hbm, idx_v, rows_v, sem):
        wid = lax.axis_index("s") * info.num_cores + lax.axis_index("c")
        base = wid * bpw
        pltpu.sync_copy(idx_hbm.at[pl.ds(base, bpw)], idx_v)
        pltpu.sync_copy(rows_hbm.at[pl.ds(base, bpw)], rows_v)
        pltpu.async_copy(rows_v, out_hbm.at[idx_v], sem).wait()

    return k(rows, idx)


def _ffn_body(ft_ref, nt_ref, ws_ref, we_ref,
              xs_ref, wi_ref, bi_ref, wg_ref, bg_ref, wo_ref, bo_ref,
              out_ref):
    h = pl.program_id(0)
    g = pl.program_id(1)
    nh = pl.num_programs(0)
    ws = ws_ref[g]
    we = we_ref[g]
    ft = ft_ref[g]
    wi = wi_ref[0]
    wg = wg_ref[0]
    wo = wo_ref[0].astype(jnp.bfloat16)

    def tile_step(i, carry):
        t = ft + i
        rows = t * TM + lax.broadcasted_iota(jnp.int32, (TM, 1), 0)
        mask = (rows >= ws) & (rows < we)
        x = xs_ref[pl.ds(t * TM, TM), :]
        proj = jnp.dot(x, wi, preferred_element_type=jnp.float32,
                       precision=lax.Precision.DEFAULT)
        proj = proj + bi_ref[pl.ds(g, 1), pl.ds(h * HK, HK)]
        gate = jnp.dot(x, wg, preferred_element_type=jnp.float32,
                       precision=lax.Precision.DEFAULT)
        gate = gate + bg_ref[pl.ds(g, 1), pl.ds(h * HK, HK)]
        u = gate * lax.logistic(gate) * proj
        um = jnp.where(mask, u, 0.0).astype(jnp.bfloat16)
        part = jnp.dot(um, wo, preferred_element_type=jnp.float32,
                       precision=lax.Precision.DEFAULT)
        sl = pl.ds(t * TM, TM)
        prev = out_ref[sl, :]

        @pl.when(h == 0)
        def _():
            out_ref[sl, :] = jnp.where(mask, part, prev)

        @pl.when(jnp.logical_and(h > 0, h < nh - 1))
        def _():
            out_ref[sl, :] = jnp.where(mask, prev + part, prev)

        @pl.when(h == nh - 1)
        def _():
            y = prev + part + bo_ref[pl.ds(g, 1), :]
            out_ref[sl, :] = jnp.where(mask, y, prev)

        return carry

    lax.fori_loop(0, nt_ref[g], tile_step, 0)


def _grouped_ffn(xs, Wi, bi, Wg, bg, Wo, bo, ft, ntl, ws, we):
    T, C = xs.shape
    E, _, H = Wi.shape
    nh = H // HK

    ys = pl.pallas_call(
        _ffn_body,
        grid_spec=pltpu.PrefetchScalarGridSpec(
            num_scalar_prefetch=4,
            grid=(nh, E),
            in_specs=[
                pl.BlockSpec((T, C), lambda h, g, *_: (0, 0)),
                pl.BlockSpec((1, C, HK), lambda h, g, *_: (g, 0, h)),
                pl.BlockSpec((E, H), lambda h, g, *_: (0, 0)),
                pl.BlockSpec((1, C, HK), lambda h, g, *_: (g, 0, h)),
                pl.BlockSpec((E, H), lambda h, g, *_: (0, 0)),
                pl.BlockSpec((1, HK, C), lambda h, g, *_: (g, h, 0)),
                pl.BlockSpec((E, C), lambda h, g, *_: (0, 0)),
            ],
            out_specs=pl.BlockSpec((T, C), lambda h, g, *_: (0, 0)),
        ),
        out_shape=jax.ShapeDtypeStruct((T, C), jnp.float32),
        compiler_params=pltpu.CompilerParams(
            dimension_semantics=("arbitrary", "arbitrary")),
    )(ft, ntl, ws, we, xs, Wi, bi, Wg, bg, Wo, bo)
    return ys


def kernel(x, Wi, bi, Wg, bg, Wo, bo, rand_maps):
    B, T_, C = x.shape
    E = Wi.shape[0]
    T = B * T_
    xf = x.reshape(T, C)

    rank, ft, ntl, ws, we = _routing_metadata(rand_maps, T, E, TM)

    xs = _sc_scatter_rows(xf, rank)
    ys = _grouped_ffn(xs, Wi, bi, Wg, bg, Wo, bo, ft, ntl, ws, we)
    outf = _sc_gather_rows(ys, rank)
    return outf.reshape(B, T_, C)


# speedup vs baseline: 1.0317x; 1.0113x over previous
import functools

import jax
import jax.numpy as jnp
from jax import lax
from jax.experimental import pallas as pl
from jax.experimental.pallas import tpu as pltpu
from jax.experimental.pallas import tpu_sc as plsc

TM = 256
HK = 1024


def _routing_metadata(rand_maps, T, E, tm):
    perm = jnp.argsort(rand_maps, stable=True).astype(jnp.int32)
    counts = jnp.bincount(rand_maps, length=E)
    ends = jnp.cumsum(counts).astype(jnp.int32)
    starts = (ends - counts).astype(jnp.int32)
    first_tile = (starts // tm).astype(jnp.int32)
    ntiles = jnp.where(counts > 0,
                       (ends + tm - 1) // tm - first_tile, 0).astype(jnp.int32)
    return perm, first_tile, ntiles, starts, ends


def _sc_gather_rows(table, idx):
    Tr, Cc = table.shape
    info = plsc.get_sparse_core_info()
    nw = info.num_cores * info.num_subcores
    bpw = Tr // nw
    mesh = plsc.VectorSubcoreMesh(core_axis_name="c", subcore_axis_name="s")

    @functools.partial(
        pl.kernel,
        mesh=mesh,
        out_type=jax.ShapeDtypeStruct((Tr, Cc), table.dtype),
        scratch_types=[
            pltpu.VMEM((bpw,), jnp.int32),
            pltpu.VMEM((bpw, Cc), table.dtype),
            pltpu.SemaphoreType.DMA,
        ],
    )
    def k(table_hbm, idx_hbm, out_hbm, idx_v, rows_v, sem):
        wid = lax.axis_index("s") * info.num_cores + lax.axis_index("c")
        base = wid * bpw
        pltpu.sync_copy(idx_hbm.at[pl.ds(base, bpw)], idx_v)
        pltpu.async_copy(table_hbm.at[idx_v], rows_v, sem).wait()
        pltpu.sync_copy(rows_v, out_hbm.at[pl.ds(base, bpw)])

    return k(table, idx)


def _sc_scatter_rows(rows, idx):
    Tr, Cc = rows.shape
    info = plsc.get_sparse_core_info()
    nw = info.num_cores * info.num_subcores
    bpw = Tr // nw
    mesh = plsc.VectorSubcoreMesh(core_axis_name="c", subcore_axis_name="s")

    @functools.partial(
        pl.kernel,
        mesh=mesh,
        out_type=jax.ShapeDtypeStruct((Tr, Cc), rows.dtype),
        scratch_types=[
            pltpu.VMEM((bpw,), jnp.int32),
            pltpu.VMEM((bpw, Cc), rows.dtype),
            pltpu.SemaphoreType.DMA,
        ],
    )
    def k(rows_hbm, idx_hbm, out_hbm, idx_v, rows_v, sem):
        wid = lax.axis_index("s") * info.num_cores + lax.axis_index("c")
        base = wid * bpw
        pltpu.sync_copy(idx_hbm.at[pl.ds(base, bpw)], idx_v)
        pltpu.sync_copy(rows_hbm.at[pl.ds(base, bpw)], rows_v)
        pltpu.async_copy(rows_v, out_hbm.at[idx_v], sem).wait()

    return k(rows, idx)


def _ffn_body(ft_ref, nt_ref, ws_ref, we_ref,
              xs_ref, wi_ref, bi_ref, wg_ref, bg_ref, wo_ref, bo_ref,
              out_ref):
    h = pl.program_id(0)
    g = pl.program_id(1)
    nh = pl.num_programs(0)
    ws = ws_ref[g]
    we = we_ref[g]
    ft = ft_ref[g]
    wi = wi_ref[0]
    wg = wg_ref[0]
    wo = wo_ref[0].astype(jnp.bfloat16)

    def tile_step(i, carry):
        t = ft + i
        rows = t * TM + lax.broadcasted_iota(jnp.int32, (TM, 1), 0)
        mask = (rows >= ws) & (rows < we)
        x = xs_ref[pl.ds(t * TM, TM), :]
        proj = jnp.dot(x, wi, preferred_element_type=jnp.float32,
                       precision=lax.Precision.DEFAULT)
        proj = proj + bi_ref[pl.ds(g, 1), pl.ds(h * HK, HK)]
        gate = jnp.dot(x, wg, preferred_element_type=jnp.float32,
                       precision=lax.Precision.DEFAULT)
        gate = gate + bg_ref[pl.ds(g, 1), pl.ds(h * HK, HK)]
        u = gate * lax.logistic(gate) * proj
        um = jnp.where(mask, u, 0.0).astype(jnp.bfloat16)
        part = jnp.dot(um, wo, preferred_element_type=jnp.float32,
                       precision=lax.Precision.DEFAULT)
        sl = pl.ds(t * TM, TM)
        prev = out_ref[sl, :]

        @pl.when(h == 0)
        def _():
            out_ref[sl, :] = jnp.where(mask, part, prev)

        @pl.when(jnp.logical_and(h > 0, h < nh - 1))
        def _():
            out_ref[sl, :] = jnp.where(mask, prev + part, prev)

        @pl.when(h == nh - 1)
        def _():
            y = prev + part + bo_ref[pl.ds(g, 1), :]
            out_ref[sl, :] = jnp.where(mask, y, prev)

        return carry

    lax.fori_loop(0, nt_ref[g], tile_step, 0)


def _grouped_ffn(xs, Wi, bi, Wg, bg, Wo, bo, ft, ntl, ws, we):
    T, C = xs.shape
    E, _, H = Wi.shape
    nh = H // HK

    ys = pl.pallas_call(
        _ffn_body,
        grid_spec=pltpu.PrefetchScalarGridSpec(
            num_scalar_prefetch=4,
            grid=(nh, E),
            in_specs=[
                pl.BlockSpec((T, C), lambda h, g, *_: (0, 0)),
                pl.BlockSpec((1, C, HK), lambda h, g, *_: (g, 0, h)),
                pl.BlockSpec((E, H), lambda h, g, *_: (0, 0)),
                pl.BlockSpec((1, C, HK), lambda h, g, *_: (g, 0, h)),
                pl.BlockSpec((E, H), lambda h, g, *_: (0, 0)),
                pl.BlockSpec((1, HK, C), lambda h, g, *_: (g, h, 0)),
                pl.BlockSpec((E, C), lambda h, g, *_: (0, 0)),
            ],
            out_specs=pl.BlockSpec((T, C), lambda h, g, *_: (0, 0)),
        ),
        out_shape=jax.ShapeDtypeStruct((T, C), jnp.float32),
        compiler_params=pltpu.CompilerParams(
            dimension_semantics=("arbitrary", "arbitrary")),
    )(ft, ntl, ws, we, xs, Wi, bi, Wg, bg, Wo, bo)
    return ys


def kernel(x, Wi, bi, Wg, bg, Wo, bo, rand_maps):
    B, T_, C = x.shape
    E = Wi.shape[0]
    T = B * T_
    xf = x.reshape(T, C)

    perm, ft, ntl, ws, we = _routing_metadata(rand_maps, T, E, TM)

    xs = _sc_gather_rows(xf, perm)
    ys = _grouped_ffn(xs, Wi, bi, Wg, bg, Wo, bo, ft, ntl, ws, we)
    outf = _sc_scatter_rows(ys, perm)
    return outf.reshape(B, T_, C)
